# baseline (speedup 1.0000x reference)
import functools

import jax
import jax.numpy as jnp
from jax import lax
from jax.experimental import pallas as pl
from jax.experimental.pallas import tpu as pltpu
from jax.experimental.pallas import tpu_sc as plsc

_B, _A = 16384, 512
_N = _B * _A
_NC, _NS, _L = 2, 16, 16
_NW = _NC * _NS
_PER_W = _N // _NW
_CHUNK = 16384
_NCHUNK = _PER_W // _CHUNK


def _body(loc_hbm, tgt_hbm, msk_hbm, out_hbm, loc_v, tgt_v, msk_v):
    wid = lax.axis_index("s") * _NC + lax.axis_index("c")
    base = wid * _PER_W

    def chunk_body(i, _):
        off = pl.multiple_of(base + i * _CHUNK, _CHUNK)
        pltpu.sync_copy(loc_hbm.at[pl.ds(off, _CHUNK)], loc_v)
        pltpu.sync_copy(tgt_hbm.at[pl.ds(off, _CHUNK)], tgt_v)
        pltpu.sync_copy(msk_hbm.at[pl.ds(off, _CHUNK)], msk_v)

        def vec_body(v, _):
            sl = pl.ds(v * _L, _L)
            l = loc_v[sl]
            t = tgt_v[sl]
            m = msk_v[sl]
            loc_v[sl] = jnp.where(m != 0, t, l)
            return _

        lax.fori_loop(0, _CHUNK // _L, vec_body, 0)
        pltpu.sync_copy(loc_v, out_hbm.at[pl.ds(off, _CHUNK)])
        return _

    lax.fori_loop(0, _NCHUNK, chunk_body, 0)


@jax.jit
def kernel(location, movement_targets, movement_mask):
    loc = location.reshape(-1)
    tgt = movement_targets.reshape(-1)
    msk = movement_mask.reshape(-1).astype(jnp.int32)
    mesh = plsc.VectorSubcoreMesh(core_axis_name="c", subcore_axis_name="s")
    out = pl.kernel(
        _body,
        mesh=mesh,
        out_type=jax.ShapeDtypeStruct((_N,), jnp.int32),
        scratch_types=[
            pltpu.VMEM((_CHUNK,), jnp.int32),
            pltpu.VMEM((_CHUNK,), jnp.int32),
            pltpu.VMEM((_CHUNK,), jnp.int32),
        ],
    )(loc, tgt, msk)
    return out.reshape(_B, _A)


# speedup vs baseline: 1.4708x; 1.4708x over previous
import functools

import jax
import jax.numpy as jnp
from jax import lax
from jax.experimental import pallas as pl
from jax.experimental.pallas import tpu as pltpu
from jax.experimental.pallas import tpu_sc as plsc

_B, _A = 16384, 512
_N = _B * _A
_NC, _NS, _L = 2, 16, 16
_NW = _NC * _NS
_PER_W = _N // _NW
_CHUNK = 16384
_NCHUNK = _PER_W // _CHUNK


def _body(loc_hbm, tgt_hbm, msk_hbm, out_hbm,
          loc_v0, loc_v1, tgt_v0, tgt_v1, msk_v0, msk_v1,
          in_sem0, in_sem1, out_sem0, out_sem1):
    wid = lax.axis_index("s") * _NC + lax.axis_index("c")
    base = wid * _PER_W
    loc_v = (loc_v0, loc_v1)
    tgt_v = (tgt_v0, tgt_v1)
    msk_v = (msk_v0, msk_v1)
    in_sem = (in_sem0, in_sem1)
    out_sem = (out_sem0, out_sem1)

    def chunk_off(i):
        return pl.multiple_of(base + i * _CHUNK, _CHUNK)

    def start_in(i, p):
        off = chunk_off(i)
        return (
            pltpu.async_copy(loc_hbm.at[pl.ds(off, _CHUNK)], loc_v[p], in_sem[p]),
            pltpu.async_copy(tgt_hbm.at[pl.ds(off, _CHUNK)], tgt_v[p], in_sem[p]),
            pltpu.async_copy(msk_hbm.at[pl.ds(off, _CHUNK)], msk_v[p], in_sem[p]),
        )

    in_descs = {0: start_in(0, 0)}
    out_descs = {}
    for i in range(_NCHUNK):
        p = i % 2
        if i + 1 < _NCHUNK:
            if i >= 1:
                out_descs.pop(i - 1).wait()
            in_descs[i + 1] = start_in(i + 1, 1 - p)
        for d in in_descs.pop(i):
            d.wait()

        lv, tv, mv = loc_v[p], tgt_v[p], msk_v[p]

        @plsc.parallel_loop(0, _CHUNK, step=_L, unroll=8)
        def vec_body(e):
            sl = pl.ds(e, _L)
            l = lv[sl]
            t = tv[sl]
            m = mv[sl]
            lv[sl] = jnp.where(m != 0, t, l)

        out_descs[i] = pltpu.async_copy(
            lv, out_hbm.at[pl.ds(chunk_off(i), _CHUNK)], out_sem[p])

    for i in sorted(out_descs):
        out_descs.pop(i).wait()


@jax.jit
def kernel(location, movement_targets, movement_mask):
    loc = location.reshape(-1)
    tgt = movement_targets.reshape(-1)
    msk = movement_mask.reshape(-1).astype(jnp.int32)
    mesh = plsc.VectorSubcoreMesh(core_axis_name="c", subcore_axis_name="s")
    out = pl.kernel(
        _body,
        mesh=mesh,
        out_type=jax.ShapeDtypeStruct((_N,), jnp.int32),
        scratch_types=[
            pltpu.VMEM((_CHUNK,), jnp.int32),
            pltpu.VMEM((_CHUNK,), jnp.int32),
            pltpu.VMEM((_CHUNK,), jnp.int32),
            pltpu.VMEM((_CHUNK,), jnp.int32),
            pltpu.VMEM((_CHUNK,), jnp.int32),
            pltpu.VMEM((_CHUNK,), jnp.int32),
            pltpu.SemaphoreType.DMA,
            pltpu.SemaphoreType.DMA,
            pltpu.SemaphoreType.DMA,
            pltpu.SemaphoreType.DMA,
        ],
    )(loc, tgt, msk)
    return out.reshape(_B, _A)


# speedup vs baseline: 3.5486x; 2.4127x over previous
import functools

import jax
import jax.numpy as jnp
from jax import lax
from jax.experimental import pallas as pl
from jax.experimental.pallas import tpu as pltpu
from jax.experimental.pallas import tpu_sc as plsc

_B, _A = 16384, 512
_NC, _NS, _L = 2, 16, 16
_NW = _NC * _NS
_ROWS_W = _B // _NW
_CR = 32
_NCHUNK = _ROWS_W // _CR
_NPAIR = _NCHUNK // 2


def _body(loc_hbm, tgt_hbm, msk_hbm, out_hbm,
          loc_v0, loc_v1, tgt_v0, tgt_v1, msk_v0, msk_v1,
          in_sem0, in_sem1, out_sem0, out_sem1):
    wid = lax.axis_index("s") * _NC + lax.axis_index("c")
    base = wid * _ROWS_W
    loc_v = (loc_v0, loc_v1)
    tgt_v = (tgt_v0, tgt_v1)
    msk_v = (msk_v0, msk_v1)
    in_sem = (in_sem0, in_sem1)
    out_sem = (out_sem0, out_sem1)

    def row_off(i):
        return pl.multiple_of(base + i * _CR, _CR)

    def in_copies(i, p):
        r0 = row_off(i)
        return (
            pltpu.make_async_copy(loc_hbm.at[pl.ds(r0, _CR), :], loc_v[p], in_sem[p]),
            pltpu.make_async_copy(tgt_hbm.at[pl.ds(r0, _CR), :], tgt_v[p], in_sem[p]),
            pltpu.make_async_copy(msk_hbm.at[pl.ds(r0, _CR), :], msk_v[p], in_sem[p]),
        )

    def start_in(i, p):
        for d in in_copies(i, p):
            d.start()

    def wait_in(i, p):
        for d in in_copies(i, p):
            d.wait()

    def out_copy(i, p):
        return pltpu.make_async_copy(
            loc_v[p], out_hbm.at[pl.ds(row_off(i), _CR), :], out_sem[p])

    def compute(p):
        lv, tv, mv = loc_v[p], tgt_v[p], msk_v[p]

        @plsc.parallel_loop(0, _CR, unroll=2)
        def row_body(r):
            for c in range(0, _A, _L):
                sl = pl.ds(c, _L)
                l = lv[r, sl]
                t = tv[r, sl]
                m = mv[r, sl]
                lv[r, sl] = jnp.where(m != 0, t, l)

    start_in(0, 0)

    def pair_body(j, carry):
        a = j * 2
        wait_in(a, 0)

        @pl.when(j > 0)
        def _():
            out_copy(a - 2, 0).wait()

        start_in(a + 1, 1)
        compute(0)
        out_copy(a, 0).start()

        wait_in(a + 1, 1)

        @pl.when(j > 0)
        def _():
            out_copy(a - 1, 1).wait()

        @pl.when(j < _NPAIR - 1)
        def _():
            start_in(a + 2, 0)

        compute(1)
        out_copy(a + 1, 1).start()
        return carry

    lax.fori_loop(0, _NPAIR, pair_body, 0)
    out_copy(_NCHUNK - 2, 0).wait()
    out_copy(_NCHUNK - 1, 1).wait()


@jax.jit
def kernel(location, movement_targets, movement_mask):
    msk = movement_mask.astype(jnp.int32)
    mesh = plsc.VectorSubcoreMesh(core_axis_name="c", subcore_axis_name="s")
    out = pl.kernel(
        _body,
        mesh=mesh,
        out_type=jax.ShapeDtypeStruct((_B, _A), jnp.int32),
        scratch_types=[
            pltpu.VMEM((_CR, _A), jnp.int32),
            pltpu.VMEM((_CR, _A), jnp.int32),
            pltpu.VMEM((_CR, _A), jnp.int32),
            pltpu.VMEM((_CR, _A), jnp.int32),
            pltpu.VMEM((_CR, _A), jnp.int32),
            pltpu.VMEM((_CR, _A), jnp.int32),
            pltpu.SemaphoreType.DMA,
            pltpu.SemaphoreType.DMA,
            pltpu.SemaphoreType.DMA,
            pltpu.SemaphoreType.DMA,
        ],
        compiler_params=pltpu.CompilerParams(use_tc_tiling_on_sc=True),
    )(location, movement_targets, msk)
    return out


# speedup vs baseline: 4.4270x; 1.2476x over previous
import functools

import jax
import jax.numpy as jnp
from jax import lax
from jax.experimental import pallas as pl
from jax.experimental.pallas import tpu as pltpu
from jax.experimental.pallas import tpu_sc as plsc

_B, _A = 16384, 512
_NC, _NS, _L = 2, 16, 16
_NW = _NC * _NS
_ROWS_W = _B // _NW
_CR = 32
_NCHUNK = _ROWS_W // _CR
_NPAIR = _NCHUNK // 2
_NGRP = (_CR // 4) * (_A // _L)


def _body(loc_hbm, tgt_hbm, msk_hbm, out_hbm,
          loc_v0, loc_v1, tgt_v0, tgt_v1, msk_v0, msk_v1,
          in_sem0, in_sem1, out_sem0, out_sem1):
    wid = lax.axis_index("s") * _NC + lax.axis_index("c")
    base = wid * _ROWS_W
    loc_v = (loc_v0, loc_v1)
    tgt_v = (tgt_v0, tgt_v1)
    msk_v = (msk_v0, msk_v1)
    in_sem = (in_sem0, in_sem1)
    out_sem = (out_sem0, out_sem1)

    def row_off(i):
        return pl.multiple_of(base + i * _CR, _CR)

    def in_copies(i, p):
        r0 = row_off(i)
        return (
            pltpu.make_async_copy(loc_hbm.at[pl.ds(r0, _CR), :], loc_v[p], in_sem[p]),
            pltpu.make_async_copy(tgt_hbm.at[pl.ds(r0, _CR), :], tgt_v[p], in_sem[p]),
            pltpu.make_async_copy(msk_hbm.at[pl.ds(r0, _CR), :], msk_v[p], in_sem[p]),
        )

    def start_in(i, p):
        for d in in_copies(i, p):
            d.start()

    def wait_in(i, p):
        for d in in_copies(i, p):
            d.wait()

    def out_copy(i, p):
        return pltpu.make_async_copy(
            loc_v[p], out_hbm.at[pl.ds(row_off(i), _CR), :], out_sem[p])

    def compute(p):
        lv, tv = loc_v[p], tgt_v[p]
        mw = msk_v[p].bitcast(jnp.int32)

        @plsc.parallel_loop(0, _NGRP, unroll=4)
        def grp_body(g):
            s = g >> 5
            c = (g & 31) * _L
            m_words = mw[s, pl.ds(c, _L)]
            for k in range(4):
                r = s * 4 + k
                sl = pl.ds(c, _L)
                m8 = lax.shift_right_logical(m_words, 8 * k) & 0xFF
                l = lv[r, sl]
                t = tv[r, sl]
                lv[r, sl] = jnp.where(m8 != 0, t, l)

    start_in(0, 0)

    def pair_body(j, carry):
        a = j * 2
        wait_in(a, 0)

        @pl.when(j > 0)
        def _():
            out_copy(a - 2, 0).wait()

        start_in(a + 1, 1)
        compute(0)
        out_copy(a, 0).start()

        wait_in(a + 1, 1)

        @pl.when(j > 0)
        def _():
            out_copy(a - 1, 1).wait()

        @pl.when(j < _NPAIR - 1)
        def _():
            start_in(a + 2, 0)

        compute(1)
        out_copy(a + 1, 1).start()
        return carry

    lax.fori_loop(0, _NPAIR, pair_body, 0)
    out_copy(_NCHUNK - 2, 0).wait()
    out_copy(_NCHUNK - 1, 1).wait()


@jax.jit
def kernel(location, movement_targets, movement_mask):
    mesh = plsc.VectorSubcoreMesh(core_axis_name="c", subcore_axis_name="s")
    out = pl.kernel(
        _body,
        mesh=mesh,
        out_type=jax.ShapeDtypeStruct((_B, _A), jnp.int32),
        scratch_types=[
            pltpu.VMEM((_CR, _A), jnp.int32),
            pltpu.VMEM((_CR, _A), jnp.int32),
            pltpu.VMEM((_CR, _A), jnp.int32),
            pltpu.VMEM((_CR, _A), jnp.int32),
            pltpu.VMEM((_CR, _A), jnp.uint8),
            pltpu.VMEM((_CR, _A), jnp.uint8),
            pltpu.SemaphoreType.DMA,
            pltpu.SemaphoreType.DMA,
            pltpu.SemaphoreType.DMA,
            pltpu.SemaphoreType.DMA,
        ],
        compiler_params=pltpu.CompilerParams(use_tc_tiling_on_sc=True),
    )(location, movement_targets, movement_mask.view(jnp.uint8))
    return out
